# baseline (speedup 1.0000x reference)
import functools

import jax
import jax.numpy as jnp
from jax import lax
from jax.experimental import pallas as pl
from jax.experimental.pallas import tpu as pltpu
from jax.experimental.pallas import tpu_sc as plsc

N = 50000
E = 800000
D_IN = 64
D_HID = 32
N_CLASS = 40

NC = 2
NS = 16
NW = NC * NS
CHUNK = 128
N_PAD_ROWS = 176
N_ACC = N + N_PAD_ROWS
ROWS_PER_TILE = N_ACC // NS
ZROWS = ROWS_PER_TILE // 8

NCH_DEG = -(-E // (NW * CHUNK))
E_PAD = NW * CHUNK * NCH_DEG
NCH_SC = E_PAD // (NC * NS * CHUNK)
IB = 14
NBLK = NCH_SC // IB
WROWS = ROWS_PER_TILE // 64

_mesh = plsc.VectorSubcoreMesh(core_axis_name="c", subcore_axis_name="s")


@functools.partial(
    pl.kernel,
    out_type=jax.ShapeDtypeStruct((NC * N_ACC,), jnp.float32),
    mesh=_mesh,
    scratch_types=[
        pltpu.VMEM((NCH_DEG * CHUNK,), jnp.int32),
        pltpu.VMEM((CHUNK,), jnp.float32),
        pltpu.VMEM((ROWS_PER_TILE,), jnp.float32),
        pltpu.VMEM_SHARED((N_ACC,), jnp.float32),
    ],
)
def _deg_kernel(dst_hbm, deg_out, dst_v, ones_v, zb_v, acc_sh):
    c = lax.axis_index("c")
    s = lax.axis_index("s")
    w = s * NC + c

    def fill(i, _):
        ones_v[pl.ds(i * 16, 16)] = jnp.ones((16,), jnp.float32)
        return 0

    lax.fori_loop(0, CHUNK // 16, fill, 0)

    def zfill(i, _):
        zb_v[pl.ds(i * 16, 16)] = jnp.zeros((16,), jnp.float32)
        return 0

    lax.fori_loop(0, ROWS_PER_TILE // 16, zfill, 0)

    pltpu.sync_copy(zb_v, acc_sh.at[pl.ds(s * ROWS_PER_TILE, ROWS_PER_TILE)])
    plsc.subcore_barrier()

    pltpu.sync_copy(dst_hbm.at[pl.ds(w * NCH_DEG * CHUNK, NCH_DEG * CHUNK)],
                    dst_v)

    def body(j, _):
        pltpu.sync_copy(
            ones_v, acc_sh.at[dst_v.at[pl.ds(j * CHUNK, CHUNK)]], add=True
        )
        return 0

    lax.fori_loop(0, NCH_DEG, body, 0)
    plsc.subcore_barrier()

    pltpu.sync_copy(acc_sh.at[pl.ds(s * ROWS_PER_TILE, ROWS_PER_TILE)], zb_v)
    pltpu.sync_copy(
        zb_v, deg_out.at[pl.ds(c * N_ACC + s * ROWS_PER_TILE, ROWS_PER_TILE)]
    )


@functools.partial(
    pl.kernel,
    out_type=jax.ShapeDtypeStruct((NC, N_ACC, D_HID), jnp.float32),
    mesh=_mesh,
    scratch_types=[
        pltpu.VMEM((2, IB * CHUNK), jnp.int32),
        pltpu.VMEM((2, IB * CHUNK), jnp.int32),
        pltpu.VMEM((4, CHUNK, D_HID), jnp.float32),
        pltpu.VMEM((2, WROWS, D_HID), jnp.float32),
        pltpu.VMEM_SHARED((N_ACC, D_HID), jnp.float32),
        pltpu.SemaphoreType.DMA,
        pltpu.SemaphoreType.DMA,
        pltpu.SemaphoreType.DMA,
        pltpu.SemaphoreType.DMA,
    ],
    compiler_params=pltpu.CompilerParams(use_tc_tiling_on_sc=False),
)
def _scatter_kernel(y_hbm, src_hbm, dst_hbm, acc_out,
                    src_v, dst_v, rows_v, zb_v, acc_sh, gsem, ssem, wsem,
                    isem):
    c = lax.axis_index("c")
    s = lax.axis_index("s")

    def zfill(i, _):
        zb_v[0, i, pl.ds(0, 16)] = jnp.zeros((16,), jnp.float32)
        zb_v[0, i, pl.ds(16, 16)] = jnp.zeros((16,), jnp.float32)
        return 0

    lax.fori_loop(0, WROWS, zfill, 0)
    for z in range(ROWS_PER_TILE // WROWS):
        pltpu.sync_copy(
            zb_v.at[0], acc_sh.at[pl.ds(s * ROWS_PER_TILE + z * WROWS, WROWS)]
        )
    plsc.subcore_barrier()

    def idx(v, p, j):
        return v.at[p, pl.ds(j * CHUNK, CHUNK)]

    def wait_gather(p, j, b):
        pltpu.make_async_copy(
            y_hbm.at[idx(src_v, p, j)], rows_v.at[b], gsem
        ).wait()

    def wait_scatter():
        pltpu.make_async_copy(
            rows_v.at[0], acc_sh.at[idx(dst_v, 0, 0)], ssem
        ).wait()

    def blk_off(blk):
        return ((c * NS + s) * NCH_SC + blk * IB) * CHUNK

    def prefetch_idx(blk, p):
        off = blk_off(blk)
        pltpu.async_copy(src_hbm.at[pl.ds(off, IB * CHUNK)], src_v.at[p],
                         isem)
        pltpu.async_copy(dst_hbm.at[pl.ds(off, IB * CHUNK)], dst_v.at[p],
                         isem)

    def wait_idx(blk, p):
        off = blk_off(blk)
        pltpu.make_async_copy(src_hbm.at[pl.ds(off, IB * CHUNK)],
                              src_v.at[p], isem).wait()
        pltpu.make_async_copy(dst_hbm.at[pl.ds(off, IB * CHUNK)],
                              dst_v.at[p], isem).wait()

    prefetch_idx(0, 0)

    def blk_body(blk, _):
        p = lax.rem(blk, 2)
        wait_idx(blk, p)
        @pl.when(blk + 1 < NBLK)
        def _():
            prefetch_idx(blk + 1, 1 - p)

        pltpu.async_copy(y_hbm.at[idx(src_v, p, 0)], rows_v.at[0], gsem)
        pltpu.async_copy(y_hbm.at[idx(src_v, p, 1)], rows_v.at[1], gsem)

        for j in range(IB):
            if j + 2 < IB:
                if j >= 2:
                    wait_scatter()
                pltpu.async_copy(
                    y_hbm.at[idx(src_v, p, j + 2)], rows_v.at[(j + 2) % 4],
                    gsem,
                )
            wait_gather(p, j, j % 4)
            pltpu.async_copy(
                rows_v.at[j % 4], acc_sh.at[idx(dst_v, p, j)], ssem, add=True
            )
        for _ in range(4):
            wait_scatter()
        return 0

    lax.fori_loop(0, NBLK, blk_body, 0)
    plsc.subcore_barrier()

    NZ = ROWS_PER_TILE // WROWS

    def wait_writeout(z):
        r0 = s * ROWS_PER_TILE + z * WROWS
        pltpu.make_async_copy(
            zb_v.at[z % 2], acc_out.at[c, pl.ds(r0, WROWS)], wsem
        ).wait()

    for z in range(NZ):
        if z >= 2:
            wait_writeout(z - 2)
        r0 = s * ROWS_PER_TILE + z * WROWS
        pltpu.sync_copy(acc_sh.at[pl.ds(r0, WROWS)], zb_v.at[z % 2])
        pltpu.async_copy(zb_v.at[z % 2], acc_out.at[c, pl.ds(r0, WROWS)], wsem)
    wait_writeout(NZ - 2)
    wait_writeout(NZ - 1)


def _dinv128(dsum4_ref, e_ref):
    d = jnp.dot(dsum4_ref[...], e_ref[...], preferred_element_type=jnp.float32)
    return lax.rsqrt(d)


def _dense1_body(x4_ref, dsum4_ref, e_ref, w4_ref, y_ref):
    xw = jnp.dot(x4_ref[...], w4_ref[...], preferred_element_type=jnp.float32)
    y_ref[...] = xw * _dinv128(dsum4_ref, e_ref)


def _dense2_body(acc_ref, y_ref, dsum4_ref, e_ref, b1_ref, w2_ref, b2_ref,
                 z_ref):
    tot = acc_ref[0] + acc_ref[1] + y_ref[...]
    h = tot * _dinv128(dsum4_ref, e_ref) + b1_ref[...]
    e = jnp.maximum(h, 0.0)
    qb = e.shape[0]
    for k in range(128 // D_HID):
        ek = e[:, k * D_HID:(k + 1) * D_HID]
        zk = (
            jnp.dot(ek, w2_ref[...], preferred_element_type=jnp.float32)
            + b2_ref[...]
        )
        z_ref[pl.Slice(k, qb, 128 // D_HID), :] = zk


RB = 6272
GSTEPS = N_ACC // RB


def kernel(x, edge_index, W1, b1, W2, b2):
    npad = E_PAD - E
    ar = jnp.arange(npad, dtype=jnp.int32)
    dst = edge_index[1].astype(jnp.int32)
    dst_p = jnp.concatenate([dst, N + (ar % N_PAD_ROWS)])
    dst_p = lax.optimization_barrier(dst_p)
    deg = _deg_kernel(dst_p)
    src = edge_index[0].astype(jnp.int32)
    src_p = jnp.concatenate([src, ar % CHUNK])

    PK = 128 // D_HID
    QB = RB // PK
    ZW = PK * N_CLASS

    dsum4 = (deg[:N_ACC] + deg[N_ACC:] + 1.0).reshape(N_ACC // PK, PK)
    rep = jnp.repeat(jnp.eye(PK, dtype=jnp.float32), D_HID, axis=1)
    x4 = x.reshape(N // PK, PK * D_IN)
    W4 = jnp.zeros((PK, D_IN, PK, D_HID), jnp.float32)
    for k in range(PK):
        W4 = W4.at[k, :, k, :].set(W1)
    W4 = W4.reshape(PK * D_IN, 128)
    b1t = jnp.tile(b1, PK).reshape(1, 128)

    y128 = pl.pallas_call(
        _dense1_body,
        grid=(GSTEPS,),
        in_specs=[
            pl.BlockSpec((QB, PK * D_IN), lambda i: (i, 0)),
            pl.BlockSpec((QB, PK), lambda i: (i, 0)),
            pl.BlockSpec((PK, 128), lambda i: (0, 0)),
            pl.BlockSpec((PK * D_IN, 128), lambda i: (0, 0)),
        ],
        out_specs=pl.BlockSpec((QB, 128), lambda i: (i, 0)),
        out_shape=jax.ShapeDtypeStruct((N_ACC // PK, 128), jnp.float32),
    )(x4, dsum4, rep, W4)

    acc = _scatter_kernel(y128.reshape(N_ACC, D_HID), src_p, dst_p)
    acc128 = acc.reshape(NC, N_ACC // PK, 128)

    z = pl.pallas_call(
        _dense2_body,
        grid=(GSTEPS,),
        in_specs=[
            pl.BlockSpec((NC, QB, 128), lambda i: (0, i, 0)),
            pl.BlockSpec((QB, 128), lambda i: (i, 0)),
            pl.BlockSpec((QB, PK), lambda i: (i, 0)),
            pl.BlockSpec((PK, 128), lambda i: (0, 0)),
            pl.BlockSpec((1, 128), lambda i: (0, 0)),
            pl.BlockSpec((D_HID, N_CLASS), lambda i: (0, 0)),
            pl.BlockSpec((1, N_CLASS), lambda i: (0, 0)),
        ],
        out_specs=pl.BlockSpec((RB, N_CLASS), lambda i: (i, 0)),
        out_shape=jax.ShapeDtypeStruct((N, N_CLASS), jnp.float32),
    )(acc128, y128, dsum4, rep, b1t, W2, b2.reshape(1, N_CLASS))

    return z
